# Initial kernel scaffold; baseline (speedup 1.0000x reference)
import jax
import jax.numpy as jnp
from jax.experimental import pallas as pl


def kernel(scores, twin_deltas):
    raise NotImplementedError("write your pallas kernel here")


# speedup vs baseline: 16.7875x; 16.7875x over previous
import functools

import jax
import jax.numpy as jnp
import numpy as np
from jax import lax
from jax.experimental import pallas as pl
from jax.experimental.pallas import tpu as pltpu

_STRIDE = 8
_SCALES = np.array([2, 4, 5, 6, 8, 9, 10, 12, 14, 16], dtype=np.float64)
_A = 10
_L = 512
_N = _L * _A
_PRE = 2000
_POST = 300
_PAD = 2048
_BK = 256
_TH = 0.7
_MIN = 8.0
_VLEN = float(_L * _STRIDE)

_W_NP = np.tile((_STRIDE * _SCALES).astype(np.float32), _L)
_C_NP = (4.0 + _STRIDE * np.repeat(np.arange(_L), _A)).astype(np.float32)


def _transform_kernel(sc_ref, dx_ref, dl_ref, w_ref, c_ref,
                      ps_ref, pe_ref, scf_ref):
    w = w_ref[...]
    c = c_ref[...]
    dx = dx_ref[...]
    dl = dl_ref[...]
    sc = sc_ref[...]
    pred_ctr = dx * w + c
    pred_l = jnp.exp(dl) * w
    ps = jnp.clip(pred_ctr - 0.5 * pred_l, 0.0, _VLEN - 1.0)
    pe = jnp.clip(pred_ctr + 0.5 * pred_l, 0.0, _VLEN - 1.0)
    ls = pe - ps + 1.0
    ps_ref[...] = ps
    pe_ref[...] = pe
    scf_ref[...] = jnp.where(ls < _MIN, 0.0, sc)


def _nms_kernel(s_col_ref, e_col_ref, s_row_ref, e_row_ref, props_ref,
                out_ref, wb_scr):
    s_col = s_col_ref[0]
    e_col = e_col_ref[0]
    s_row = s_row_ref[0]
    e_row = e_row_ref[0]
    props = props_ref[0]

    len_col = e_col - s_col + 1.0
    len_row = e_row - s_row + 1.0

    lane = lax.broadcasted_iota(jnp.int32, (1, _BK), 1)
    tri = (lax.broadcasted_iota(jnp.int32, (_BK, _BK), 0)
           <= lax.broadcasted_iota(jnp.int32, (_BK, _BK), 1)
           ).astype(jnp.float32)

    keep_blocks = []
    rank_blocks = []
    offset = jnp.zeros((1, 1), jnp.float32)
    zero_bk = jnp.zeros((1, _BK), jnp.float32)

    for b in range(_PAD // _BK):
        j0 = b * _BK
        sr = lax.slice(s_row, (0, j0), (1, j0 + _BK))
        er = lax.slice(e_row, (0, j0), (1, j0 + _BK))
        lr = lax.slice(len_row, (0, j0), (1, j0 + _BK))
        inter = jnp.clip(jnp.minimum(e_col, er) - jnp.maximum(s_col, sr)
                         + 1.0, 0.0, None)
        union = len_col + lr - inter
        m = (inter / union > _TH).astype(jnp.float32)

        keep_prev = jnp.concatenate(
            keep_blocks + [zero_bk] * (_PAD // _BK - b), axis=1)
        cnt = jnp.dot(keep_prev, m, preferred_element_type=jnp.float32)
        valid = (lane + j0 < _PRE).astype(jnp.float32)
        local = (cnt == 0.0).astype(jnp.float32) * valid

        wb_scr[...] = lax.slice(m, (j0, 0), (j0 + _BK, _BK))

        def body(k, lk):
            row = wb_scr[pl.ds(k, 1), :]
            lkk = jnp.sum(jnp.where(lane == k, lk, 0.0))
            sup = (row > 0.0) & (lane > k) & (lkk > 0.0)
            return lk * (1.0 - sup.astype(jnp.float32))

        local = lax.fori_loop(0, _BK, body, local)
        keep_blocks.append(local)

        prefix = jnp.dot(local, tri, preferred_element_type=jnp.float32)
        rank_blocks.append(prefix + offset)
        offset = offset + lax.slice(prefix, (0, _BK - 1), (1, _BK))

    keep = jnp.concatenate(keep_blocks, axis=1)
    rank = jnp.concatenate(rank_blocks, axis=1)

    s_idx = lax.broadcasted_iota(jnp.int32, (_POST, _PAD), 0).astype(
        jnp.float32)
    onehot = ((rank == s_idx + 1.0) & (keep > 0.0)).astype(jnp.float32)
    out_ref[0] = jnp.dot(onehot, props, preferred_element_type=jnp.float32)


@jax.jit
def kernel(scores, twin_deltas):
    B = scores.shape[0]
    sc = jnp.transpose(scores.reshape(B, 2 * _A, _L)[:, _A:, :],
                       (0, 2, 1)).reshape(B, _N)
    d = jnp.transpose(twin_deltas.reshape(B, 2 * _A, _L),
                      (0, 2, 1)).reshape(B, _N, 2)
    dx = d[..., 0]
    dl = d[..., 1]
    w = jnp.asarray(_W_NP)[None, :]
    c = jnp.asarray(_C_NP)[None, :]

    ps, pe, scf = pl.pallas_call(
        _transform_kernel,
        out_shape=[jax.ShapeDtypeStruct((B, _N), jnp.float32)] * 3,
    )(sc, dx, dl, w, c)

    _, idx = jax.lax.top_k(scf, _PRE)
    ps_s = jnp.take_along_axis(ps, idx, axis=1)
    pe_s = jnp.take_along_axis(pe, idx, axis=1)
    pad = ((0, 0), (0, _PAD - _PRE))
    ps_p = jnp.pad(ps_s, pad)
    pe_p = jnp.pad(pe_s, pad)
    props = jnp.stack([ps_p, pe_p], axis=-1)

    out = pl.pallas_call(
        _nms_kernel,
        grid=(B,),
        in_specs=[
            pl.BlockSpec((1, _PAD, 1), lambda b: (b, 0, 0)),
            pl.BlockSpec((1, _PAD, 1), lambda b: (b, 0, 0)),
            pl.BlockSpec((1, 1, _PAD), lambda b: (b, 0, 0)),
            pl.BlockSpec((1, 1, _PAD), lambda b: (b, 0, 0)),
            pl.BlockSpec((1, _PAD, 2), lambda b: (b, 0, 0)),
        ],
        out_specs=pl.BlockSpec((1, _POST, 2), lambda b: (b, 0, 0)),
        out_shape=jax.ShapeDtypeStruct((B, _POST, 2), jnp.float32),
        scratch_shapes=[pltpu.VMEM((_BK, _BK), jnp.float32)],
    )(ps_p[:, :, None], pe_p[:, :, None], ps_p[:, None, :],
      pe_p[:, None, :], props)

    bi = jnp.broadcast_to(
        jnp.arange(B, dtype=jnp.float32)[:, None, None], (B, _POST, 1))
    return jnp.concatenate([bi, out], axis=2)


# speedup vs baseline: 107.7171x; 6.4165x over previous
import functools

import jax
import jax.numpy as jnp
import numpy as np
from jax import lax
from jax.experimental import pallas as pl
from jax.experimental.pallas import tpu as pltpu

_STRIDE = 8
_SCALES = np.array([2, 4, 5, 6, 8, 9, 10, 12, 14, 16], dtype=np.float64)
_A = 10
_L = 512
_N = _L * _A
_PRE = 2000
_POST = 300
_PAD = 2048
_BK = 256
_TH = 0.7
_MIN = 8.0
_VLEN = float(_L * _STRIDE)

_W_NP = np.tile((_STRIDE * _SCALES).astype(np.float32), _L)
_C_NP = (4.0 + _STRIDE * np.repeat(np.arange(_L), _A)).astype(np.float32)


def _transform_kernel(sc_ref, dx_ref, dl_ref, w_ref, c_ref,
                      ps_ref, pe_ref, scf_ref):
    w = w_ref[...]
    c = c_ref[...]
    dx = dx_ref[...]
    dl = dl_ref[...]
    sc = sc_ref[...]
    pred_ctr = dx * w + c
    pred_l = jnp.exp(dl) * w
    ps = jnp.clip(pred_ctr - 0.5 * pred_l, 0.0, _VLEN - 1.0)
    pe = jnp.clip(pred_ctr + 0.5 * pred_l, 0.0, _VLEN - 1.0)
    ls = pe - ps + 1.0
    ps_ref[...] = ps
    pe_ref[...] = pe
    scf_ref[...] = jnp.where(ls < _MIN, 0.0, sc)


_B = 8


def _nms_kernel(s_col_ref, e_col_ref, s_row_ref, e_row_ref, props_ref,
                out_ref, wb_scr):
    s_col = s_col_ref[...]
    e_col = e_col_ref[...]
    s_row = s_row_ref[...]
    e_row = e_row_ref[...]
    len_col = e_col - s_col + 1.0
    len_row = e_row - s_row + 1.0

    lane = lax.broadcasted_iota(jnp.int32, (1, _BK), 1)
    lane8 = lax.broadcasted_iota(jnp.int32, (_B, _BK), 1)
    io0 = lax.broadcasted_iota(jnp.int32, (_BK, _BK), 0)
    io1 = lax.broadcasted_iota(jnp.int32, (_BK, _BK), 1)
    tri = (io0 <= io1).astype(jnp.float32)
    tri_strict = (io0 < io1).astype(jnp.float32)

    keep_blocks = [[] for _ in range(_B)]
    rank_blocks = [[] for _ in range(_B)]
    offsets = [jnp.zeros((1, 1), jnp.float32)] * _B

    for b in range(_PAD // _BK):
        j0 = b * _BK
        rows0 = j0 + _BK
        valid = (lane + j0 < _PRE).astype(jnp.float32)
        locals_list = []
        wb_list = []
        for bb in range(_B):
            s_c = s_col[bb, :rows0, :]
            e_c = e_col[bb, :rows0, :]
            l_c = len_col[bb, :rows0, :]
            sr = s_row[bb, :, j0:rows0]
            er = e_row[bb, :, j0:rows0]
            lr = len_row[bb, :, j0:rows0]
            inter = jnp.clip(jnp.minimum(e_c, er) - jnp.maximum(s_c, sr)
                             + 1.0, 0.0, None)
            union = l_c + lr - inter
            m = (inter / union > _TH).astype(jnp.float32)
            if b == 0:
                local_b = valid
            else:
                keep_prev = jnp.concatenate(keep_blocks[bb], axis=1)
                cnt = jnp.dot(keep_prev, m[:j0, :],
                              preferred_element_type=jnp.float32)
                local_b = (cnt == 0.0).astype(jnp.float32) * valid
            locals_list.append(local_b)
            wb_list.append((m[j0:rows0, :] * tri_strict)[None])
        wb_scr[...] = jnp.concatenate(wb_list, axis=0)
        lk = jnp.concatenate(locals_list, axis=0)

        def body(k, lk):
            rows = wb_scr[:, pl.ds(k, 1), :].reshape(_B, _BK)
            lkk = jnp.sum(jnp.where(lane8 == k, lk, 0.0), axis=1,
                          keepdims=True)
            sup = (rows > 0.0) & (lkk > 0.0)
            return lk * (1.0 - sup.astype(jnp.float32))

        lk = lax.fori_loop(0, _BK, body, lk)

        for bb in range(_B):
            local_b = lax.slice(lk, (bb, 0), (bb + 1, _BK))
            keep_blocks[bb].append(local_b)
            prefix = jnp.dot(local_b, tri,
                             preferred_element_type=jnp.float32)
            rank_blocks[bb].append(prefix + offsets[bb])
            offsets[bb] = offsets[bb] + lax.slice(
                prefix, (0, _BK - 1), (1, _BK))

    s_idx = lax.broadcasted_iota(jnp.int32, (_POST, _PAD), 0).astype(
        jnp.float32)
    for bb in range(_B):
        keep_b = jnp.concatenate(keep_blocks[bb], axis=1)
        rank_b = jnp.concatenate(rank_blocks[bb], axis=1)
        onehot = ((rank_b == s_idx + 1.0) & (keep_b > 0.0)).astype(
            jnp.float32)
        out_ref[bb] = jnp.dot(onehot, props_ref[bb],
                              preferred_element_type=jnp.float32)


@jax.jit
def kernel(scores, twin_deltas):
    B = scores.shape[0]
    sc = jnp.transpose(scores.reshape(B, 2 * _A, _L)[:, _A:, :],
                       (0, 2, 1)).reshape(B, _N)
    d = jnp.transpose(twin_deltas.reshape(B, 2 * _A, _L),
                      (0, 2, 1)).reshape(B, _N, 2)
    dx = d[..., 0]
    dl = d[..., 1]
    w = jnp.asarray(_W_NP)[None, :]
    c = jnp.asarray(_C_NP)[None, :]

    ps, pe, scf = pl.pallas_call(
        _transform_kernel,
        out_shape=[jax.ShapeDtypeStruct((B, _N), jnp.float32)] * 3,
    )(sc, dx, dl, w, c)

    _, idx = jax.lax.top_k(scf, _PRE)
    ps_s = jnp.take_along_axis(ps, idx, axis=1)
    pe_s = jnp.take_along_axis(pe, idx, axis=1)
    pad = ((0, 0), (0, _PAD - _PRE))
    ps_p = jnp.pad(ps_s, pad)
    pe_p = jnp.pad(pe_s, pad)
    props = jnp.stack([ps_p, pe_p], axis=-1)

    out = pl.pallas_call(
        _nms_kernel,
        out_shape=jax.ShapeDtypeStruct((B, _POST, 2), jnp.float32),
        scratch_shapes=[pltpu.VMEM((_B, _BK, _BK), jnp.float32)],
    )(ps_p[:, :, None], pe_p[:, :, None], ps_p[:, None, :],
      pe_p[:, None, :], props)

    bi = jnp.broadcast_to(
        jnp.arange(B, dtype=jnp.float32)[:, None, None], (B, _POST, 1))
    return jnp.concatenate([bi, out], axis=2)
